# baseline (speedup 1.0000x reference)
import functools

import jax
import jax.numpy as jnp
from jax import lax
from jax.experimental import pallas as pl
from jax.experimental.pallas import tpu as pltpu
from jax.experimental.pallas import tpu_sc as plsc

N = 10000
N_PAD = 10240
E = 320000
D = 128

NC = 2
NS = 16
NW = NC * NS
EPW = 10240
E_PAD = NW * EPW
IROWS = EPW // 128
CH = 64
NB = CH // 16
ROWS_PER_TILE = N_PAD // NS
ZROWS = 64

_mesh = plsc.VectorSubcoreMesh(core_axis_name="c", subcore_axis_name="s")


def _fill(buf, nrows, val):
    @pl.loop(0, nrows)
    def _(r):
        for j in range(D // 16):
            buf[r, pl.ds(j * 16, 16)] = jnp.full((16,), val, jnp.float32)


def _zero_acc_slab(zbuf, acc, s):
    for j in range(ROWS_PER_TILE // ZROWS):
        pltpu.sync_copy(
            zbuf, acc.at[pl.ds(s * ROWS_PER_TILE + j * ZROWS, ZROWS)])


def _copy_out_slab(acc, zbuf, out_hbm, c, s):
    for j in range(ROWS_PER_TILE // ZROWS):
        row0 = s * ROWS_PER_TILE + j * ZROWS
        pltpu.sync_copy(acc.at[pl.ds(row0, ZROWS)], zbuf)
        pltpu.sync_copy(zbuf, out_hbm.at[c, pl.ds(row0, ZROWS)])


@functools.partial(
    pl.kernel,
    out_type=jax.ShapeDtypeStruct((NC, N_PAD, D), jnp.float32),
    mesh=_mesh,
    scratch_types=[
        pltpu.VMEM((IROWS, 128), jnp.int32),
        pltpu.VMEM((CH, D), jnp.float32),
        pltpu.VMEM_SHARED((N_PAD, D), jnp.float32),
        pltpu.SemaphoreType.DMA,
    ],
)
def _sc_degree(dst_hbm, out_hbm, idx_d2, buf, acc, ssem):
    c = lax.axis_index("c")
    s = lax.axis_index("s")
    wid = c * NS + s

    _fill(buf, CH, 0.0)
    _zero_acc_slab(buf, acc, s)
    pltpu.sync_copy(dst_hbm.at[wid], idx_d2)
    _fill(buf, CH, 1.0)
    plsc.subcore_barrier()

    @pl.loop(0, IROWS)
    def _(r):
        for b in range(8):
            idxv = idx_d2[r, pl.ds(b * 16, 16)]
            pltpu.sync_copy(buf.at[pl.ds((b % NB) * 16, 16)], acc.at[idxv],
                            add=True)

    plsc.subcore_barrier()
    _copy_out_slab(acc, buf, out_hbm, c, s)


@functools.partial(
    pl.kernel,
    out_type=jax.ShapeDtypeStruct((NC, N_PAD, D), jnp.float32),
    mesh=_mesh,
    scratch_types=[
        pltpu.VMEM((IROWS, 128), jnp.int32),
        pltpu.VMEM((IROWS, 128), jnp.int32),
        pltpu.VMEM((CH, D), jnp.float32),
        pltpu.VMEM((CH, D), jnp.float32),
        pltpu.VMEM_SHARED((N_PAD, D), jnp.float32),
        pltpu.SemaphoreType.DMA,
        pltpu.SemaphoreType.DMA,
    ],
)
def _sc_aggregate(u_hbm, src_hbm, dst_hbm, out_hbm, idx_s2, idx_d2, rows0,
                  rows1, acc, gsem0, gsem1):
    c = lax.axis_index("c")
    s = lax.axis_index("s")
    wid = c * NS + s

    def gather(r, col, rows, gsem):
        descs = []
        for b in range(NB):
            idxv = idx_s2[r, pl.ds(col + b * 16, 16)]
            descs.append(pltpu.async_copy(
                u_hbm.at[idxv], rows.at[pl.ds(b * 16, 16)], gsem))
        return descs

    def scatter(r, col, rows):
        for b in range(NB):
            idxv = idx_d2[r, pl.ds(col + b * 16, 16)]
            pltpu.sync_copy(rows.at[pl.ds(b * 16, 16)], acc.at[idxv],
                            add=True)

    _fill(rows0, CH, 0.0)
    _zero_acc_slab(rows0, acc, s)
    pltpu.sync_copy(src_hbm.at[wid], idx_s2)
    pltpu.sync_copy(dst_hbm.at[wid], idx_d2)
    plsc.subcore_barrier()

    @pl.loop(0, IROWS)
    def _(r):
        ga = gather(r, 0, rows0, gsem0)
        gb = gather(r, CH, rows1, gsem1)
        for d in ga:
            d.wait()
        scatter(r, 0, rows0)
        for d in gb:
            d.wait()
        scatter(r, CH, rows1)

    plsc.subcore_barrier()
    _copy_out_slab(acc, rows0, out_hbm, c, s)


def _tc_pre(x, W1, degp):
    def body(x_ref, w_ref, degp_ref, u_ref, dinv_ref):
        deg = degp_ref[0, 0:N, 0:1] + degp_ref[1, 0:N, 0:1] + 1.0
        dinv = lax.rsqrt(deg)
        h = jnp.dot(x_ref[...], w_ref[...], preferred_element_type=jnp.float32)
        u_ref[...] = h * dinv
        dinv_ref[...] = dinv

    return pl.pallas_call(
        body,
        out_shape=(
            jax.ShapeDtypeStruct((N, D), jnp.float32),
            jax.ShapeDtypeStruct((N, 1), jnp.float32),
        ),
    )(x, W1, degp)


def _bn_relu(z, g, be, r):
    mean = jnp.mean(z, axis=0, keepdims=True)
    zc = z - mean
    var = jnp.mean(zc * zc, axis=0, keepdims=True)
    zb = zc * lax.rsqrt(var + 1e-5) * g + be
    if r is not None:
        zb = zb + r
    return jnp.maximum(zb, 0.0)


def _tc_mid(accp, u, dinv, g, be, Wn, r=None):
    has_r = r is not None

    def body(*refs):
        if has_r:
            (accp_ref, u_ref, dinv_ref, g_ref, be_ref, w_ref, r_ref, un_ref,
             a_ref) = refs
        else:
            accp_ref, u_ref, dinv_ref, g_ref, be_ref, w_ref, un_ref, a_ref = refs
        z = (accp_ref[0, 0:N, :] + accp_ref[1, 0:N, :] + u_ref[...]) * dinv_ref[...]
        a = _bn_relu(z, g_ref[...], be_ref[...],
                     r_ref[...] if has_r else None)
        a_ref[...] = a
        un_ref[...] = jnp.dot(
            a, w_ref[...], preferred_element_type=jnp.float32) * dinv_ref[...]

    args = [accp, u, dinv, g.reshape(1, D), be.reshape(1, D), Wn]
    if has_r:
        args.append(r)
    return pl.pallas_call(
        body,
        out_shape=(
            jax.ShapeDtypeStruct((N, D), jnp.float32),
            jax.ShapeDtypeStruct((N, D), jnp.float32),
        ),
    )(*args)


def _tc_final(accp, u, dinv, g, be, Wh, bh, r):
    def body(accp_ref, u_ref, dinv_ref, g_ref, be_ref, wh_ref, bh_ref, r_ref,
             out_ref):
        z = (accp_ref[0, 0:N, :] + accp_ref[1, 0:N, :] + u_ref[...]) * dinv_ref[...]
        a = _bn_relu(z, g_ref[...], be_ref[...], r_ref[...])
        out_ref[...] = jnp.dot(
            a, wh_ref[...], preferred_element_type=jnp.float32) + bh_ref[...]

    nl = Wh.shape[1]
    return pl.pallas_call(
        body,
        out_shape=jax.ShapeDtypeStruct((N, nl), jnp.float32),
    )(accp, u, dinv, g.reshape(1, D), be.reshape(1, D), Wh, bh.reshape(1, nl),
      r)


def kernel(x, edge_index, W1, b1, W2, b2, W3, b3, g1, be1, g2, be2, g3, be3,
           Wh, bh):
    src = edge_index[0].astype(jnp.int32)
    dst = edge_index[1].astype(jnp.int32)
    pad = E_PAD - E
    src = jnp.concatenate([src, jnp.zeros((pad,), jnp.int32)])
    fill_dst = N + jnp.arange(pad, dtype=jnp.int32) % (N_PAD - N)
    dst = jnp.concatenate([dst, fill_dst])
    src = src.reshape(NW, IROWS, 128)
    dst = dst.reshape(NW, IROWS, 128)

    degp = _sc_degree(dst)
    u1, dinv = _tc_pre(x, W1, degp)

    agg1 = _sc_aggregate(u1, src, dst)
    u2, a1 = _tc_mid(agg1, u1, dinv, g1, be1, W2)

    agg2 = _sc_aggregate(u2, src, dst)
    u3, a2 = _tc_mid(agg2, u2, dinv, g2, be2, W3, r=a1)

    agg3 = _sc_aggregate(u3, src, dst)
    logits = _tc_final(agg3, u3, dinv, g3, be3, Wh, bh, r=a2)
    return logits


# speedup vs baseline: 2.1547x; 2.1547x over previous
import functools

import jax
import jax.numpy as jnp
from jax import lax
from jax.experimental import pallas as pl
from jax.experimental.pallas import tpu as pltpu
from jax.experimental.pallas import tpu_sc as plsc

N = 10000
N_PAD = 10240
E = 320000
D = 128

NC = 2
NS = 16
NW = NC * NS
EPW = 10240
E_PAD = NW * EPW
IROWS = EPW // 128
CH = 64
NB = CH // 16
ROWS_PER_TILE = N_PAD // NS
ZROWS = 64

_mesh = plsc.VectorSubcoreMesh(core_axis_name="c", subcore_axis_name="s")


def _fill(buf, nrows, val):
    @pl.loop(0, nrows)
    def _(r):
        for j in range(D // 16):
            buf[r, pl.ds(j * 16, 16)] = jnp.full((16,), val, jnp.float32)


def _zero_acc_slab(zbuf, acc, s):
    for j in range(ROWS_PER_TILE // ZROWS):
        pltpu.sync_copy(
            zbuf, acc.at[pl.ds(s * ROWS_PER_TILE + j * ZROWS, ZROWS)])


def _copy_out_slab(acc, zbuf, out_hbm, c, s):
    for j in range(ROWS_PER_TILE // ZROWS):
        row0 = s * ROWS_PER_TILE + j * ZROWS
        pltpu.sync_copy(acc.at[pl.ds(row0, ZROWS)], zbuf)
        pltpu.sync_copy(zbuf, out_hbm.at[c, pl.ds(row0, ZROWS)])


@functools.partial(
    pl.kernel,
    out_type=jax.ShapeDtypeStruct((NC, N_PAD, D), jnp.float32),
    mesh=_mesh,
    scratch_types=[
        pltpu.VMEM((IROWS, 128), jnp.int32),
        pltpu.VMEM((CH, D), jnp.float32),
        pltpu.VMEM_SHARED((N_PAD, D), jnp.float32),
        pltpu.SemaphoreType.DMA,
    ],
)
def _sc_degree(dst_hbm, out_hbm, idx_d2, buf, acc, ssem):
    c = lax.axis_index("c")
    s = lax.axis_index("s")
    wid = c * NS + s

    _fill(buf, CH, 0.0)
    _zero_acc_slab(buf, acc, s)
    pltpu.sync_copy(dst_hbm.at[wid], idx_d2)
    _fill(buf, CH, 1.0)
    plsc.subcore_barrier()

    @pl.loop(0, IROWS)
    def _(r):
        for b in range(8):
            idxv = idx_d2[r, pl.ds(b * 16, 16)]
            pltpu.sync_copy(buf.at[pl.ds((b % NB) * 16, 16)], acc.at[idxv],
                            add=True)

    plsc.subcore_barrier()
    _copy_out_slab(acc, buf, out_hbm, c, s)


@functools.partial(
    pl.kernel,
    out_type=jax.ShapeDtypeStruct((NC, N_PAD, D), jnp.float32),
    mesh=_mesh,
    scratch_types=[
        pltpu.VMEM((IROWS, 128), jnp.int32),
        pltpu.VMEM((IROWS, 128), jnp.int32),
        pltpu.VMEM((CH, D), jnp.float32),
        pltpu.VMEM((CH, D), jnp.float32),
        pltpu.VMEM_SHARED((N_PAD, D), jnp.float32),
        pltpu.SemaphoreType.DMA,
        pltpu.SemaphoreType.DMA,
    ],
)
def _sc_aggregate(u_hbm, src_hbm, dst_hbm, out_hbm, idx_s2, idx_d2, rows0,
                  rows1, acc, gsem0, gsem1):
    c = lax.axis_index("c")
    s = lax.axis_index("s")
    wid = c * NS + s

    def gather(r, col, rows, gsem):
        descs = []
        for b in range(NB):
            idxv = idx_s2[r, pl.ds(col + b * 16, 16)]
            descs.append(pltpu.async_copy(
                u_hbm.at[idxv], rows.at[pl.ds(b * 16, 16)], gsem))
        return descs

    def scatter(r, col, rows):
        for b in range(NB):
            idxv = idx_d2[r, pl.ds(col + b * 16, 16)]
            pltpu.sync_copy(rows.at[pl.ds(b * 16, 16)], acc.at[idxv],
                            add=True)

    _fill(rows0, CH, 0.0)
    _zero_acc_slab(rows0, acc, s)
    pltpu.sync_copy(src_hbm.at[wid], idx_s2)
    pltpu.sync_copy(dst_hbm.at[wid], idx_d2)
    plsc.subcore_barrier()

    @pl.loop(0, IROWS)
    def _(r):
        ga = gather(r, 0, rows0, gsem0)
        gb = gather(r, CH, rows1, gsem1)
        for d in ga:
            d.wait()
        scatter(r, 0, rows0)
        for d in gb:
            d.wait()
        scatter(r, CH, rows1)

    plsc.subcore_barrier()
    _copy_out_slab(acc, rows0, out_hbm, c, s)


def _tc_pre(x, W1, degp):
    def body(x_ref, w_ref, degp_ref, u_ref, dinv_ref):
        deg = degp_ref[0, 0:N, 0:1] + degp_ref[1, 0:N, 0:1] + 1.0
        dinv = lax.rsqrt(deg)
        h = jnp.dot(x_ref[...], w_ref[...], preferred_element_type=jnp.float32)
        u_ref[...] = h * dinv
        dinv_ref[...] = dinv

    return pl.pallas_call(
        body,
        out_shape=(
            jax.ShapeDtypeStruct((N, D), jnp.float32),
            jax.ShapeDtypeStruct((N, 1), jnp.float32),
        ),
    )(x, W1, degp)


def _bn_relu(z, g, be, r):
    mean = jnp.mean(z, axis=0, keepdims=True)
    zc = z - mean
    var = jnp.mean(zc * zc, axis=0, keepdims=True)
    zb = zc * lax.rsqrt(var + 1e-5) * g + be
    if r is not None:
        zb = zb + r
    return jnp.maximum(zb, 0.0)


def _tc_mid(accp, u, dinv, g, be, Wn, r=None):
    has_r = r is not None

    def body(*refs):
        if has_r:
            (accp_ref, u_ref, dinv_ref, g_ref, be_ref, w_ref, r_ref, un_ref,
             a_ref) = refs
        else:
            accp_ref, u_ref, dinv_ref, g_ref, be_ref, w_ref, un_ref, a_ref = refs
        z = (accp_ref[0, 0:N, :] + accp_ref[1, 0:N, :] + u_ref[...]) * dinv_ref[...]
        a = _bn_relu(z, g_ref[...], be_ref[...],
                     r_ref[...] if has_r else None)
        a_ref[...] = a
        un_ref[...] = jnp.dot(
            a, w_ref[...], preferred_element_type=jnp.float32) * dinv_ref[...]

    args = [accp, u, dinv, g.reshape(1, D), be.reshape(1, D), Wn]
    if has_r:
        args.append(r)
    return pl.pallas_call(
        body,
        out_shape=(
            jax.ShapeDtypeStruct((N, D), jnp.float32),
            jax.ShapeDtypeStruct((N, D), jnp.float32),
        ),
    )(*args)


def _tc_final(accp, u, dinv, g, be, Wh, bh, r):
    def body(accp_ref, u_ref, dinv_ref, g_ref, be_ref, wh_ref, bh_ref, r_ref,
             out_ref):
        z = (accp_ref[0, 0:N, :] + accp_ref[1, 0:N, :] + u_ref[...]) * dinv_ref[...]
        a = _bn_relu(z, g_ref[...], be_ref[...], r_ref[...])
        out_ref[...] = jnp.dot(
            a, wh_ref[...], preferred_element_type=jnp.float32) + bh_ref[...]

    nl = Wh.shape[1]
    return pl.pallas_call(
        body,
        out_shape=jax.ShapeDtypeStruct((N, nl), jnp.float32),
    )(accp, u, dinv, g.reshape(1, D), be.reshape(1, D), Wh, bh.reshape(1, nl),
      r)


def kernel(x, edge_index, W1, b1, W2, b2, W3, b3, g1, be1, g2, be2, g3, be3,
           Wh, bh):
    src = edge_index[0].astype(jnp.int32)
    dst = edge_index[1].astype(jnp.int32)
    ppt = EPW - E // NW
    dummy_src = jnp.broadcast_to(jnp.arange(ppt, dtype=jnp.int32), (NW, ppt))
    dummy_dst = N + dummy_src % (N_PAD - N)
    src = jnp.concatenate([src.reshape(NW, E // NW), dummy_src], axis=1)
    dst = jnp.concatenate([dst.reshape(NW, E // NW), dummy_dst], axis=1)
    src = src.reshape(NW, IROWS, 128)
    dst = dst.reshape(NW, IROWS, 128)

    degp = _sc_degree(dst)
    u1, dinv = _tc_pre(x, W1, degp)

    agg1 = _sc_aggregate(u1, src, dst)
    u2, a1 = _tc_mid(agg1, u1, dinv, g1, be1, W2)

    agg2 = _sc_aggregate(u2, src, dst)
    u3, a2 = _tc_mid(agg2, u2, dinv, g2, be2, W3, r=a1)

    agg3 = _sc_aggregate(u3, src, dst)
    logits = _tc_final(agg3, u3, dinv, g3, be3, Wh, bh, r=a2)
    return logits
